# baseline (speedup 1.0000x reference)
import functools

import jax
import jax.numpy as jnp
from jax import lax
from jax.experimental import pallas as pl
from jax.experimental.pallas import tpu as pltpu
from jax.experimental.pallas import tpu_sc as plsc


def _sc_gather_bf16(table, ids):
    n = ids.shape[0]
    h = table.shape[1]
    hw = h // 2
    info = plsc.get_sparse_core_info()
    nw = info.num_cores * info.num_subcores
    per_w = n // nw
    ch = 16
    n_ch = per_w // ch
    ng = 4
    no = 4
    mesh = plsc.VectorSubcoreMesh(core_axis_name="c", subcore_axis_name="s")

    @functools.partial(
        pl.kernel,
        out_type=jax.ShapeDtypeStruct((n, hw), jnp.float32),
        mesh=mesh,
        compiler_params=pltpu.CompilerParams(needs_layout_passes=False),
        scratch_types=[
            pltpu.VMEM((per_w,), jnp.int32),
            *[pltpu.VMEM((ch, h), jnp.float32) for _ in range(ng)],
            *[pltpu.VMEM((ch, hw), jnp.float32) for _ in range(no)],
            *[pltpu.SemaphoreType.DMA for _ in range(ng + no)],
        ],
    )
    def k(table_hbm, ids_hbm, out_hbm, idx_v, *bufs_sems):
        bufs = bufs_sems[:ng]
        obufs = bufs_sems[ng:ng + no]
        gsems = bufs_sems[ng + no:2 * ng + no]
        wsems = bufs_sems[2 * ng + no:]
        wid = lax.axis_index("s") * info.num_cores + lax.axis_index("c")
        base = wid * per_w
        pltpu.sync_copy(ids_hbm.at[pl.ds(base, per_w)], idx_v)
        gathers = [None] * ng
        writes = [None] * no

        mask_hi = jnp.full((16,), 0xFFFF0000, jnp.uint32)
        shift16 = jnp.full((16,), 16, jnp.uint32)

        def pack_chunk(buf, obuf):
            def row(r, _):
                def grp(kk, _):
                    a = plsc.bitcast(buf[r, pl.ds(16 * kk, 16)], jnp.uint32)
                    b = plsc.bitcast(buf[r, pl.ds(hw + 16 * kk, 16)],
                                     jnp.uint32)
                    w = (a >> shift16) | (b & mask_hi)
                    obuf[r, pl.ds(16 * kk, 16)] = plsc.bitcast(
                        w, jnp.float32)
                    return 0

                return lax.fori_loop(0, hw // 16, grp, 0, unroll=8)

            lax.fori_loop(0, ch, row, 0)

        for i in range(min(ng, n_ch)):
            gathers[i] = pltpu.async_copy(
                table_hbm.at[idx_v.at[pl.ds(i * ch, ch)]], bufs[i], gsems[i])
        for i in range(n_ch):
            g = i % ng
            o = i % no
            gathers[g].wait()
            if writes[o] is not None:
                writes[o].wait()
            pack_chunk(bufs[g], obufs[o])
            writes[o] = pltpu.async_copy(
                obufs[o], out_hbm.at[pl.ds(base + i * ch, ch)], wsems[o])
            if i + ng < n_ch:
                gathers[g] = pltpu.async_copy(
                    table_hbm.at[idx_v.at[pl.ds((i + ng) * ch, ch)]],
                    bufs[g], gsems[g])
        for i in range(max(n_ch - no, 0), n_ch):
            writes[i % no].wait()

    return k(table, ids)


def _tc_combine(slice_i, nslices, outbuf, gp, var_idx, res_idx, lt_idx,
                st8, cat_t, w1p, b1, w2, b2):
    ns, hw = gp.shape
    n = ns * nslices
    h = 2 * hw
    tb = 2048
    nblk = ns // tb
    blk0 = slice_i * nblk
    vc = cat_t.shape[0]
    std = w2.shape[0]

    vi3 = var_idx.reshape(nblk, 1, tb)
    ri3 = res_idx.reshape(nblk, 1, tb)
    li3 = lt_idx.reshape(nblk, 1, tb)

    def body(gref, vref, rref, lref, stref, ctref,
             w1ref, b1ref, w2ref, b2ref, oref):
        vi = vref[0, 0, :]
        ri = rref[0, 0, :] + 128
        li = lref[0, 0, :] + 144
        col = lax.broadcasted_iota(jnp.int32, (tb, vc), 1)
        mh = ((vi[:, None] == col) | (ri[:, None] == col) | (li[:, None] == col)
              ).astype(jnp.bfloat16)
        u = lax.bitcast_convert_type(gref[...], jnp.uint32)
        lo = lax.bitcast_convert_type(u << 16, jnp.float32)
        hi = lax.bitcast_convert_type(u & jnp.uint32(0xFFFF0000), jnp.float32)
        mhb = mh
        ctb = ctref[...].astype(jnp.bfloat16)
        hmid = jax.nn.gelu(
            jnp.dot(stref[...], w1ref[...], preferred_element_type=jnp.float32)
            + b1ref[0])
        hb = hmid.astype(jnp.bfloat16)
        w2b = w2ref[...].astype(jnp.bfloat16)
        acc_lo = lo + jnp.dot(mhb, ctb[:, :hw],
                              preferred_element_type=jnp.float32)
        acc_lo += jnp.dot(hb, w2b[:, :hw], preferred_element_type=jnp.float32)
        acc_lo += b2ref[0, :hw]
        oref[:, :hw] = acc_lo
        acc_hi = hi + jnp.dot(mhb, ctb[:, hw:],
                              preferred_element_type=jnp.float32)
        acc_hi += jnp.dot(hb, w2b[:, hw:], preferred_element_type=jnp.float32)
        acc_hi += b2ref[0, hw:]
        oref[:, hw:] = acc_hi

    specs = [
        pl.BlockSpec((tb, hw), lambda i: (i, 0)),
        pl.BlockSpec((1, 1, tb), lambda i: (i, 0, 0)),
        pl.BlockSpec((1, 1, tb), lambda i: (i, 0, 0)),
        pl.BlockSpec((1, 1, tb), lambda i: (i, 0, 0)),
        pl.BlockSpec((tb, 8), lambda i: (i, 0)),
        pl.BlockSpec((vc, h), lambda i: (0, 0)),
        pl.BlockSpec((8, std), lambda i: (0, 0)),
        pl.BlockSpec((1, std), lambda i: (0, 0)),
        pl.BlockSpec((std, h), lambda i: (0, 0)),
        pl.BlockSpec((1, h), lambda i: (0, 0)),
    ]
    args = (gp, vi3, ri3, li3, st8, cat_t, w1p, b1, w2, b2)
    aliases = {}
    if slice_i > 0:
        specs = [pl.BlockSpec(memory_space=pl.ANY)] + specs
        args = (outbuf,) + args
        aliases = {0: 0}

    def wrapped(*refs):
        if slice_i > 0:
            body(*refs[1:])
        else:
            body(*refs)

    return pl.pallas_call(
        wrapped,
        grid=(nblk,),
        in_specs=specs,
        out_specs=pl.BlockSpec((tb, h), lambda i: (i + blk0, 0)),
        out_shape=jax.ShapeDtypeStruct((n, h), jnp.float32),
        input_output_aliases=aliases,
    )(*args)


def kernel(input_ids, position_ids, var_idx, res_idx, leadtime_idx,
           spatial_temporal_features, token_table, var_table, res_table,
           leadtime_table, W1, b1, W2, b2):
    n = input_ids.size
    h = token_table.shape[1]
    ids = input_ids.reshape(n)
    vi = var_idx.reshape(n)
    ri = res_idx.reshape(n)
    li = leadtime_idx.reshape(n)
    st8 = jnp.pad(spatial_temporal_features.reshape(n, -1), ((0, 0), (0, 1)))
    w1p = jnp.pad(W1, ((0, 1), (0, 0)))
    cat_t = jnp.concatenate([var_table, res_table, leadtime_table], axis=0)
    b1r = b1.reshape(1, -1)
    b2r = b2.reshape(1, -1)

    nsl = 1
    sl = n // nsl
    gps = [_sc_gather_bf16(token_table,
                           lax.dynamic_slice_in_dim(ids, i * sl, sl))
           for i in range(nsl)]
    out = None
    for i in range(nsl):
        s0 = i * sl
        out = _tc_combine(i, nsl, out, gps[i],
                          lax.dynamic_slice_in_dim(vi, s0, sl),
                          lax.dynamic_slice_in_dim(ri, s0, sl),
                          lax.dynamic_slice_in_dim(li, s0, sl),
                          lax.dynamic_slice_in_dim(st8, s0, sl),
                          cat_t, w1p, b1r, W2, b2r)
    return (out, position_ids)


# speedup vs baseline: 1.5097x; 1.5097x over previous
import functools

import jax
import jax.numpy as jnp
from jax import lax
from jax.experimental import pallas as pl
from jax.experimental.pallas import tpu as pltpu
from jax.experimental.pallas import tpu_sc as plsc


def _sc_gather_bf16(table, ids):
    n = ids.shape[0]
    h = table.shape[1]
    hw = h // 2
    info = plsc.get_sparse_core_info()
    nw = info.num_cores * info.num_subcores
    per_w = n // nw
    ch = 32
    n_ch = per_w // ch
    nb = 3
    mesh = plsc.VectorSubcoreMesh(core_axis_name="c", subcore_axis_name="s")

    @functools.partial(
        pl.kernel,
        out_type=jax.ShapeDtypeStruct((n, hw), jnp.float32),
        mesh=mesh,
        compiler_params=pltpu.CompilerParams(needs_layout_passes=False),
        scratch_types=[
            pltpu.VMEM((per_w,), jnp.int32),
            *[pltpu.VMEM((ch, h), jnp.float32) for _ in range(nb)],
            *[pltpu.SemaphoreType.DMA for _ in range(2 * nb)],
        ],
    )
    def k(table_hbm, ids_hbm, out_hbm, idx_v, *bufs_sems):
        bufs = bufs_sems[:nb]
        gsems = bufs_sems[nb:2 * nb]
        wsems = bufs_sems[2 * nb:]
        wid = lax.axis_index("s") * info.num_cores + lax.axis_index("c")
        base = wid * per_w
        pltpu.sync_copy(ids_hbm.at[pl.ds(base, per_w)], idx_v)
        gathers = [None] * nb
        writes = [None] * nb

        mask_hi = jnp.full((16,), 0xFFFF0000, jnp.uint32)
        shift16 = jnp.full((16,), 16, jnp.uint32)

        def pack_chunk(buf):
            def row(r, _):
                def grp(kk, _):
                    a = plsc.bitcast(buf[r, pl.ds(16 * kk, 16)], jnp.uint32)
                    b = plsc.bitcast(buf[r, pl.ds(hw + 16 * kk, 16)],
                                     jnp.uint32)
                    w = (a >> shift16) | (b & mask_hi)
                    buf[r, pl.ds(16 * kk, 16)] = plsc.bitcast(w, jnp.float32)
                    return 0

                return lax.fori_loop(0, hw // 16, grp, 0, unroll=8)

            lax.fori_loop(0, ch, row, 0)

        for i in range(min(nb, n_ch)):
            gathers[i] = pltpu.async_copy(
                table_hbm.at[idx_v.at[pl.ds(i * ch, ch)]], bufs[i], gsems[i])
        for i in range(n_ch):
            s = i % nb
            gathers[s].wait()
            pack_chunk(bufs[s])
            writes[s] = pltpu.async_copy(
                bufs[s].at[:, pl.ds(0, hw)],
                out_hbm.at[pl.ds(base + i * ch, ch)], wsems[s])
            if i + nb < n_ch:
                writes[s].wait()
                gathers[s] = pltpu.async_copy(
                    table_hbm.at[idx_v.at[pl.ds((i + nb) * ch, ch)]],
                    bufs[s], gsems[s])
        for i in range(max(n_ch - nb, 0), n_ch):
            writes[i % nb].wait()

    return k(table, ids)


def _tc_combine(slice_i, nslices, outbuf, gp, var_idx, res_idx, lt_idx,
                st8, cat_t, w1p, b1, w2, b2):
    ns, hw = gp.shape
    n = ns * nslices
    h = 2 * hw
    tb = 2048
    nblk = ns // tb
    blk0 = slice_i * nblk
    vc = cat_t.shape[0]
    std = w2.shape[0]

    vi3 = var_idx.reshape(nblk, 1, tb)
    ri3 = res_idx.reshape(nblk, 1, tb)
    li3 = lt_idx.reshape(nblk, 1, tb)

    def body(gref, vref, rref, lref, stref, ctref,
             w1ref, b1ref, w2ref, b2ref, oref):
        vi = vref[0, 0, :]
        ri = rref[0, 0, :] + 128
        li = lref[0, 0, :] + 144
        col = lax.broadcasted_iota(jnp.int32, (tb, vc), 1)
        mh = ((vi[:, None] == col) | (ri[:, None] == col) | (li[:, None] == col)
              ).astype(jnp.bfloat16)
        u = lax.bitcast_convert_type(gref[...], jnp.uint32)
        lo = lax.bitcast_convert_type(u << 16, jnp.float32)
        hi = lax.bitcast_convert_type(u & jnp.uint32(0xFFFF0000), jnp.float32)
        mhb = mh
        ctb = ctref[...].astype(jnp.bfloat16)
        hmid = jax.nn.gelu(
            jnp.dot(stref[...], w1ref[...], preferred_element_type=jnp.float32)
            + b1ref[0])
        hb = hmid.astype(jnp.bfloat16)
        w2b = w2ref[...].astype(jnp.bfloat16)
        acc_lo = lo + jnp.dot(mhb, ctb[:, :hw],
                              preferred_element_type=jnp.float32)
        acc_lo += jnp.dot(hb, w2b[:, :hw], preferred_element_type=jnp.float32)
        acc_lo += b2ref[0, :hw]
        oref[:, :hw] = acc_lo
        acc_hi = hi + jnp.dot(mhb, ctb[:, hw:],
                              preferred_element_type=jnp.float32)
        acc_hi += jnp.dot(hb, w2b[:, hw:], preferred_element_type=jnp.float32)
        acc_hi += b2ref[0, hw:]
        oref[:, hw:] = acc_hi

    specs = [
        pl.BlockSpec((tb, hw), lambda i: (i, 0)),
        pl.BlockSpec((1, 1, tb), lambda i: (i, 0, 0)),
        pl.BlockSpec((1, 1, tb), lambda i: (i, 0, 0)),
        pl.BlockSpec((1, 1, tb), lambda i: (i, 0, 0)),
        pl.BlockSpec((tb, 8), lambda i: (i, 0)),
        pl.BlockSpec((vc, h), lambda i: (0, 0)),
        pl.BlockSpec((8, std), lambda i: (0, 0)),
        pl.BlockSpec((1, std), lambda i: (0, 0)),
        pl.BlockSpec((std, h), lambda i: (0, 0)),
        pl.BlockSpec((1, h), lambda i: (0, 0)),
    ]
    args = (gp, vi3, ri3, li3, st8, cat_t, w1p, b1, w2, b2)
    aliases = {}
    if slice_i > 0:
        specs = [pl.BlockSpec(memory_space=pl.ANY)] + specs
        args = (outbuf,) + args
        aliases = {0: 0}

    def wrapped(*refs):
        if slice_i > 0:
            body(*refs[1:])
        else:
            body(*refs)

    return pl.pallas_call(
        wrapped,
        grid=(nblk,),
        in_specs=specs,
        out_specs=pl.BlockSpec((tb, h), lambda i: (i + blk0, 0)),
        out_shape=jax.ShapeDtypeStruct((n, h), jnp.float32),
        input_output_aliases=aliases,
    )(*args)


def kernel(input_ids, position_ids, var_idx, res_idx, leadtime_idx,
           spatial_temporal_features, token_table, var_table, res_table,
           leadtime_table, W1, b1, W2, b2):
    n = input_ids.size
    h = token_table.shape[1]
    ids = input_ids.reshape(n)
    vi = var_idx.reshape(n)
    ri = res_idx.reshape(n)
    li = leadtime_idx.reshape(n)
    st8 = jnp.pad(spatial_temporal_features.reshape(n, -1), ((0, 0), (0, 1)))
    w1p = jnp.pad(W1, ((0, 1), (0, 0)))
    cat_t = jnp.concatenate([var_table, res_table, leadtime_table], axis=0)
    b1r = b1.reshape(1, -1)
    b2r = b2.reshape(1, -1)

    nsl = 1
    sl = n // nsl
    gps = [_sc_gather_bf16(token_table,
                           lax.dynamic_slice_in_dim(ids, i * sl, sl))
           for i in range(nsl)]
    out = None
    for i in range(nsl):
        s0 = i * sl
        out = _tc_combine(i, nsl, out, gps[i],
                          lax.dynamic_slice_in_dim(vi, s0, sl),
                          lax.dynamic_slice_in_dim(ri, s0, sl),
                          lax.dynamic_slice_in_dim(li, s0, sl),
                          lax.dynamic_slice_in_dim(st8, s0, sl),
                          cat_t, w1p, b1r, W2, b2r)
    return (out, position_ids)
